# baseline (speedup 1.0000x reference)
import functools

import jax
import jax.numpy as jnp
from jax import lax
from jax.experimental import pallas as pl
from jax.experimental.pallas import tpu as pltpu
from jax.experimental.pallas import tpu_sc as plsc

_L = 16
_B = 128


@functools.partial(jax.jit, static_argnums=(3,))
def _probe_gather(ftab, px, py, planes):
    rows, w = ftab.shape
    h = rows // planes
    p_count = px.shape[0]
    info = plsc.get_sparse_core_info()
    nw = info.num_cores * info.num_subcores
    ppw = p_count // nw
    pchunks = planes // _L
    half = planes // 2

    mesh = plsc.VectorSubcoreMesh(core_axis_name="c", subcore_axis_name="s")

    @functools.partial(
        pl.kernel,
        out_type=jax.ShapeDtypeStruct((p_count, planes), jnp.float32),
        mesh=mesh,
        compiler_params=pltpu.CompilerParams(needs_layout_passes=False),
        scratch_types=[
            pltpu.VMEM((p_count + _L,), jnp.int32),
            pltpu.VMEM((p_count + _L,), jnp.int32),
            pltpu.VMEM((ppw * 2, half), jnp.int32),
            pltpu.VMEM((3 * planes, _B), jnp.float32),
            pltpu.VMEM((ppw, planes), jnp.float32),
            pltpu.SemaphoreType.DMA,
            pltpu.SemaphoreType.DMA,
            pltpu.SemaphoreType.DMA,
        ],
    )
    def k(ftab_hbm, px_hbm, py_hbm, out_hbm, px_v, py_v, ridx_v,
          gran_v, out_v, sem0, sem1, sem2):
        wid = lax.axis_index("s") * info.num_cores + lax.axis_index("c")
        base = wid * ppw
        pltpu.sync_copy(px_hbm, px_v.at[pl.ds(0, p_count)])
        pltpu.sync_copy(py_hbm, py_v.at[pl.ds(0, p_count)])

        lane = lax.iota(jnp.int32, _L)
        sems = [sem0, sem1, sem2]

        def fire(j):
            buf = j % 3
            sx = px_v[pl.ds(base + j, _L)][0]
            sy = py_v[pl.ds(base + j, _L)][0]
            ystart = pl.multiple_of((sy >> 7) << 7, _B)
            hc = pchunks // 2

            def idx_body(c, _, j=j, sx=sx):
                pv = lane + c * _L
                ridx_v[j * 2 + c // hc, pl.ds((c % hc) * _L, _L)] = pv * h + sx
                return 0

            lax.fori_loop(0, pchunks, idx_body, 0)
            cps = []
            for hb in range(2):
                cp = pltpu.make_async_copy(
                    ftab_hbm.at[ridx_v.at[j * 2 + hb], pl.ds(ystart, _B)],
                    gran_v.at[pl.ds((buf * 2 + hb) * half, half)],
                    sems[buf],
                )
                cp.start()
                cps.append(cp)
            return cps

        def select(j, cps):
            buf = j % 3
            for cp in cps:
                cp.wait()
            sy = py_v[pl.ds(base + j, _L)][0]
            off = jnp.full((_L,), sy & (_B - 1), jnp.int32)

            def sel_body(c, _, j=j, buf=buf, off=off):
                rbase = buf * planes + c * _L
                out_v[j, pl.ds(c * _L, _L)] = plsc.load_gather(
                    gran_v, [lane + rbase, off]
                )
                return 0

            lax.fori_loop(0, pchunks, sel_body, 0)

        depth = 3
        pend = [fire(j) for j in range(min(depth, ppw))]
        for j in range(ppw):
            select(j, pend[j])
            if j + depth < ppw:
                pend.append(fire(j + depth))

        pltpu.sync_copy(out_v, out_hbm.at[pl.ds(base, ppw)])

    return k(ftab, px, py)


def kernel(field, probe_x, probe_y):
    b, ch, h, w = field.shape
    p_count = probe_x.shape[0]
    planes = b * ch
    ftab = field.reshape(planes * h, w)
    px = probe_x.astype(jnp.int32)
    py = probe_y.astype(jnp.int32)
    out = _probe_gather(ftab, px, py, planes)
    return out.T.reshape(b, ch, p_count)
